# Initial kernel scaffold; baseline (speedup 1.0000x reference)
import jax
import jax.numpy as jnp
from jax.experimental import pallas as pl


def kernel(x, edge_index, edge_attr, batch, We, be, W1, b1, W2, b2, gamma, beta, Wv1, bv1, Wv2, bv2, Wc, bc):
    raise NotImplementedError("write your pallas kernel here")


# speedup vs baseline: 1.1057x; 1.1057x over previous
import functools

import jax
import jax.numpy as jnp
from jax import lax
from jax.experimental import pallas as pl
from jax.experimental.pallas import tpu as pltpu
from jax.experimental.pallas import tpu_sc as plsc

N = 10000
E = 160000
D = 256
H = 128
DE = 16
L = 3
G = 128
T = 10

E_PAD = 163840
EP_SUB = E_PAD // 16
C = 256
NCH = EP_SUB // C
NSPLIT = 5120
ANR = 5376
AR_Z = ANR // 16
AR_F = NSPLIT // 16
A_OUT = 2 * NSPLIT
DPAD = 3 * NSPLIT

RC = 2000
NB = N // RC


def _sc_body(hlo, hhi, elo, ehi, src2d, dst2d, agg_lo, agg_hi,
             ebuf, mbuf, sidx, didx, acc, sem):
    c = lax.axis_index("c")
    s = lax.axis_index("s")

    def _zero_acc():
        def _zrow(r, carry):
            for j in range(H // 16):
                mbuf[r, pl.ds(j * 16, 16)] = jnp.zeros((16,), jnp.float32)
            return carry

        lax.fori_loop(0, C, _zrow, 0)
        zbase = s * AR_Z
        off = 0
        while off < AR_Z:
            step = min(C, AR_Z - off)
            pltpu.sync_copy(mbuf.at[pl.ds(0, step)],
                            acc.at[pl.ds(zbase + off, step)])
            off += step

    def _process(h_tab, e_tab, p):
        def _super(ks, carry):
            irow = s * (EP_SUB // 128) + ks * 8
            pltpu.sync_copy(src2d.at[pl.ds(irow, 8)], sidx)
            pltpu.sync_copy(dst2d.at[pl.ds(irow, 8)], didx)

            def _remap(r, carry2):
                for j in range(8):
                    sl = pl.ds(j * 16, 16)
                    d = didx[r, sl] - (p * NSPLIT)
                    ok = (d >= 0) & (d < NSPLIT)
                    didx[r, sl] = jnp.where(ok, d, NSPLIT)
                return carry2

            lax.fori_loop(0, 8, _remap, 0)

            def _chunk(q, qc):
                ebase = s * EP_SUB + ks * 1024 + q * C
                pltpu.sync_copy(e_tab.at[pl.ds(ebase, C)], ebuf)
                cps = []
                for j in range(C // 128):
                    cps.append(pltpu.async_copy(
                        h_tab.at[sidx.at[2 * q + j]],
                        mbuf.at[pl.ds(j * 128, 128)], sem))
                for cp in cps:
                    cp.wait()

                def _row(r, rc):
                    for j in range(H // 16):
                        sl = pl.ds(j * 16, 16)
                        mbuf[r, sl] = jnp.maximum(mbuf[r, sl] + ebuf[r, sl], 0.0)
                    return rc

                lax.fori_loop(0, C, _row, 0)
                for j in range(C // 128):
                    pltpu.sync_copy(mbuf.at[pl.ds(j * 128, 128)],
                                    acc.at[didx.at[2 * q + j]], add=True)
                return qc

            lax.fori_loop(0, 4, _chunk, 0)
            return carry

        lax.fori_loop(0, EP_SUB // 1024, _super, 0)

    for p in range(2):
        _zero_acc()
        plsc.subcore_barrier()

        @pl.when(c == 0)
        def _():
            _process(hlo, elo, p)

        @pl.when(c == 1)
        def _():
            _process(hhi, ehi, p)

        plsc.subcore_barrier()
        dst_row = p * NSPLIT + s * AR_F

        @pl.when(c == 0)
        def _():
            pltpu.sync_copy(acc.at[pl.ds(s * AR_F, AR_F)],
                            agg_lo.at[pl.ds(dst_row, AR_F)])

        @pl.when(c == 1)
        def _():
            pltpu.sync_copy(acc.at[pl.ds(s * AR_F, AR_F)],
                            agg_hi.at[pl.ds(dst_row, AR_F)])

        plsc.subcore_barrier()


_sc_msgpass = functools.partial(
    pl.kernel,
    mesh=plsc.VectorSubcoreMesh(core_axis_name="c", subcore_axis_name="s"),
    out_type=[
        jax.ShapeDtypeStruct((A_OUT, H), jnp.float32),
        jax.ShapeDtypeStruct((A_OUT, H), jnp.float32),
    ],
    scratch_types=[
        pltpu.VMEM((C, H), jnp.float32),
        pltpu.VMEM((C, H), jnp.float32),
        pltpu.VMEM((8, 128), jnp.int32),
        pltpu.VMEM((8, 128), jnp.int32),
        pltpu.VMEM_SHARED((ANR, H), jnp.float32),
        pltpu.SemaphoreType.DMA,
    ],
)(_sc_body)


def _enc_body(ea_ref, we_ref, be_ref, lo_ref, hi_ref):
    e = jnp.dot(ea_ref[...], we_ref[...],
                preferred_element_type=jnp.float32) + be_ref[...]
    lo_ref[...] = e[:, :H]
    hi_ref[...] = e[:, H:]


def _encode(ea_pad, we, be):
    ch = 8192
    return pl.pallas_call(
        _enc_body,
        grid=(E_PAD // ch,),
        in_specs=[
            pl.BlockSpec((ch, DE), lambda i: (i, 0)),
            pl.BlockSpec((DE, D), lambda i: (0, 0)),
            pl.BlockSpec((1, D), lambda i: (0, 0)),
        ],
        out_specs=[
            pl.BlockSpec((ch, H), lambda i: (i, 0)),
            pl.BlockSpec((ch, H), lambda i: (i, 0)),
        ],
        out_shape=[jax.ShapeDtypeStruct((E_PAD, H), jnp.float32)] * 2,
    )(ea_pad, we, be)


def _hin_body(h_ref, b_ref, vn_ref, lo_ref, hi_ref):
    bvec = b_ref[0, 0, :]
    oh = (bvec[:, None] == lax.broadcasted_iota(jnp.int32, (RC, G), 1)
          ).astype(jnp.float32)
    hin = h_ref[...] + jnp.dot(oh, vn_ref[...],
                               preferred_element_type=jnp.float32)
    lo_ref[...] = hin[:, :H]
    hi_ref[...] = hin[:, H:]


def _hin_split(h, batch3, vn):
    return pl.pallas_call(
        _hin_body,
        grid=(NB,),
        in_specs=[
            pl.BlockSpec((RC, D), lambda i: (i, 0)),
            pl.BlockSpec((1, 1, RC), lambda i: (i, 0, 0)),
            pl.BlockSpec((G, D), lambda i: (0, 0)),
        ],
        out_specs=[
            pl.BlockSpec((RC, H), lambda i: (i, 0)),
            pl.BlockSpec((RC, H), lambda i: (i, 0)),
        ],
        out_shape=[jax.ShapeDtypeStruct((N, H), jnp.float32)] * 2,
    )(h, batch3, vn)


def _pool_body(h_ref, b_ref, vn_ref, wv1_ref, bv1_ref, wv2_ref, bv2_ref,
               out_ref, acc_ref):
    i = pl.program_id(0)

    @pl.when(i == 0)
    def _():
        acc_ref[...] = jnp.zeros_like(acc_ref)

    bvec = b_ref[0, 0, :]
    oht = (lax.broadcasted_iota(jnp.int32, (G, RC), 0) == bvec[None, :]
           ).astype(jnp.float32)
    acc_ref[...] += jnp.dot(oht, h_ref[...], preferred_element_type=jnp.float32)

    @pl.when(i == pl.num_programs(0) - 1)
    def _():
        pooled = acc_ref[...] + vn_ref[...]
        t = jnp.maximum(jnp.dot(pooled, wv1_ref[...],
                                preferred_element_type=jnp.float32)
                        + bv1_ref[...], 0.0)
        v = jnp.dot(t, wv2_ref[...],
                    preferred_element_type=jnp.float32) + bv2_ref[...]
        out_ref[...] = jnp.maximum(v, 0.0)


def _pool_vn(h, batch3, vn, wv1, bv1, wv2, bv2):
    return pl.pallas_call(
        _pool_body,
        grid=(NB,),
        in_specs=[
            pl.BlockSpec((RC, D), lambda i: (i, 0)),
            pl.BlockSpec((1, 1, RC), lambda i: (i, 0, 0)),
            pl.BlockSpec((G, D), lambda i: (0, 0)),
            pl.BlockSpec((D, 2 * D), lambda i: (0, 0)),
            pl.BlockSpec((1, 2 * D), lambda i: (0, 0)),
            pl.BlockSpec((2 * D, D), lambda i: (0, 0)),
            pl.BlockSpec((1, D), lambda i: (0, 0)),
        ],
        out_specs=pl.BlockSpec((G, D), lambda i: (0, 0)),
        out_shape=jax.ShapeDtypeStruct((G, D), jnp.float32),
        scratch_shapes=[pltpu.VMEM((G, D), jnp.float32)],
    )(h, batch3, vn, wv1, bv1, wv2, bv2)


def _mlp_body(hlo_ref, hhi_ref, alo_ref, ahi_ref, w1_ref, b1_ref, w2_ref,
              b2_ref, z2_ref, stats_ref, ssum_ref, ssq_ref):
    i = pl.program_id(0)

    @pl.when(i == 0)
    def _():
        ssum_ref[...] = jnp.zeros_like(ssum_ref)
        ssq_ref[...] = jnp.zeros_like(ssq_ref)

    hin = jnp.concatenate([hlo_ref[...], hhi_ref[...]], axis=1)
    z = hin + jnp.concatenate([alo_ref[...], ahi_ref[...]], axis=1)
    t = jnp.maximum(jnp.dot(z, w1_ref[...],
                            preferred_element_type=jnp.float32)
                    + b1_ref[...], 0.0)
    z2 = jnp.dot(t, w2_ref[...], preferred_element_type=jnp.float32) + b2_ref[...]
    z2_ref[...] = z2
    ssum_ref[...] += jnp.sum(z2, axis=0, keepdims=True)
    ssq_ref[...] += jnp.sum(z2 * z2, axis=0, keepdims=True)

    @pl.when(i == pl.num_programs(0) - 1)
    def _():
        mean = ssum_ref[...] / N
        var = ssq_ref[...] / N - mean * mean
        inv = lax.rsqrt(var + 1e-5)
        stats_ref[...] = jnp.concatenate([mean, inv], axis=0)


def _mlp_stats(hlo, hhi, alo, ahi, w1, b1, w2, b2):
    return pl.pallas_call(
        _mlp_body,
        grid=(NB,),
        in_specs=[
            pl.BlockSpec((RC, H), lambda i: (i, 0)),
            pl.BlockSpec((RC, H), lambda i: (i, 0)),
            pl.BlockSpec((RC, H), lambda i: (i, 0)),
            pl.BlockSpec((RC, H), lambda i: (i, 0)),
            pl.BlockSpec((D, 2 * D), lambda i: (0, 0)),
            pl.BlockSpec((1, 2 * D), lambda i: (0, 0)),
            pl.BlockSpec((2 * D, D), lambda i: (0, 0)),
            pl.BlockSpec((1, D), lambda i: (0, 0)),
        ],
        out_specs=[
            pl.BlockSpec((RC, D), lambda i: (i, 0)),
            pl.BlockSpec((2, D), lambda i: (0, 0)),
        ],
        out_shape=[
            jax.ShapeDtypeStruct((N, D), jnp.float32),
            jax.ShapeDtypeStruct((2, D), jnp.float32),
        ],
        scratch_shapes=[pltpu.VMEM((1, D), jnp.float32),
                        pltpu.VMEM((1, D), jnp.float32)],
    )(hlo, hhi, alo, ahi, w1, b1, w2, b2)


def _norm_body(z2_ref, hlo_ref, hhi_ref, stats_ref, g_ref, bt_ref, out_ref,
               *, relu):
    hin = jnp.concatenate([hlo_ref[...], hhi_ref[...]], axis=1)
    mean = stats_ref[0:1, :]
    inv = stats_ref[1:2, :]
    zn = (z2_ref[...] - mean) * inv * g_ref[...] + bt_ref[...]
    if relu:
        zn = jnp.maximum(zn, 0.0)
    out_ref[...] = hin + zn


def _norm_residual(z2, hlo, hhi, stats, g, bt, relu):
    return pl.pallas_call(
        functools.partial(_norm_body, relu=relu),
        grid=(NB,),
        in_specs=[
            pl.BlockSpec((RC, D), lambda i: (i, 0)),
            pl.BlockSpec((RC, H), lambda i: (i, 0)),
            pl.BlockSpec((RC, H), lambda i: (i, 0)),
            pl.BlockSpec((2, D), lambda i: (0, 0)),
            pl.BlockSpec((1, D), lambda i: (0, 0)),
            pl.BlockSpec((1, D), lambda i: (0, 0)),
        ],
        out_specs=pl.BlockSpec((RC, D), lambda i: (i, 0)),
        out_shape=jax.ShapeDtypeStruct((N, D), jnp.float32),
    )(z2, hlo, hhi, stats, g, bt)


def _readout_body(h_ref, b_ref, wc_ref, bc_ref, out_ref, acc_ref):
    i = pl.program_id(0)

    @pl.when(i == 0)
    def _():
        acc_ref[...] = jnp.zeros_like(acc_ref)

    bvec = b_ref[0, 0, :]
    oht = (lax.broadcasted_iota(jnp.int32, (G, RC), 0) == bvec[None, :]
           ).astype(jnp.float32)
    acc_ref[...] += jnp.dot(oht, h_ref[...], preferred_element_type=jnp.float32)

    @pl.when(i == pl.num_programs(0) - 1)
    def _():
        out_ref[...] = jnp.dot(acc_ref[...], wc_ref[...],
                               preferred_element_type=jnp.float32) + bc_ref[...]


def _readout(h, batch3, wc_pad, bc_pad):
    return pl.pallas_call(
        _readout_body,
        grid=(NB,),
        in_specs=[
            pl.BlockSpec((RC, D), lambda i: (i, 0)),
            pl.BlockSpec((1, 1, RC), lambda i: (i, 0, 0)),
            pl.BlockSpec((D, 128), lambda i: (0, 0)),
            pl.BlockSpec((1, 128), lambda i: (0, 0)),
        ],
        out_specs=pl.BlockSpec((G, 128), lambda i: (0, 0)),
        out_shape=jax.ShapeDtypeStruct((G, 128), jnp.float32),
        scratch_shapes=[pltpu.VMEM((G, D), jnp.float32)],
    )(h, batch3, wc_pad, bc_pad)


def kernel(x, edge_index, edge_attr, batch, We, be, W1, b1, W2, b2,
           gamma, beta, Wv1, bv1, Wv2, bv2, Wc, bc):
    pad = E_PAD - E
    src = edge_index[0].astype(jnp.int32)
    dst = edge_index[1].astype(jnp.int32)
    src2d = jnp.concatenate(
        [src, jnp.zeros((pad,), jnp.int32)]).reshape(E_PAD // 128, 128)
    dst2d = jnp.concatenate(
        [dst, jnp.full((pad,), DPAD, jnp.int32)]).reshape(E_PAD // 128, 128)
    ea_pad = jnp.concatenate(
        [edge_attr, jnp.zeros((pad, DE), jnp.float32)], axis=0)
    batch3 = batch.astype(jnp.int32).reshape(NB, 1, RC)
    wc_pad = jnp.pad(Wc, ((0, 0), (0, 128 - T)))
    bc_pad = jnp.pad(bc, (0, 128 - T)).reshape(1, 128)

    h = x
    vn = jnp.zeros((G, D), jnp.float32)
    for l in range(L):
        e_lo, e_hi = _encode(ea_pad, We[l], be[l].reshape(1, D))
        if l == 0:
            h_lo = x[:, :H]
            h_hi = x[:, H:]
        else:
            vn = _pool_vn(h, batch3, vn, Wv1[l - 1], bv1[l - 1].reshape(1, 2 * D),
                          Wv2[l - 1], bv2[l - 1].reshape(1, D))
            h_lo, h_hi = _hin_split(h, batch3, vn)
        agg_lo, agg_hi = _sc_msgpass(h_lo, h_hi, e_lo, e_hi, src2d, dst2d)
        z2, stats = _mlp_stats(h_lo, h_hi, agg_lo, agg_hi, W1[l],
                               b1[l].reshape(1, 2 * D), W2[l],
                               b2[l].reshape(1, D))
        h = _norm_residual(z2, h_lo, h_hi, stats, gamma[l].reshape(1, D),
                           beta[l].reshape(1, D), relu=(l < L - 1))
    outp = _readout(h, batch3, wc_pad, bc_pad)
    return outp[:, :T]
